# Initial kernel scaffold; baseline (speedup 1.0000x reference)
import jax
import jax.numpy as jnp
from jax.experimental import pallas as pl


def kernel(x, edge_index, W, b):
    raise NotImplementedError("write your pallas kernel here")


# speedup vs baseline: 18.5766x; 18.5766x over previous
import functools

import jax
import jax.numpy as jnp
from jax import lax
from jax.experimental import pallas as pl
from jax.experimental.pallas import tpu as pltpu
from jax.experimental.pallas import tpu_sc as plsc

N = 10000
D = 128
C = 40
CP = 48
E = 320000

NC = 2
NS = 16
NW = NC * NS
BATCH = 128
EPB = ((E + NW * BATCH * 8 - 1) // (NW * BATCH * 8)) * 8
E_PAD = NW * BATCH * EPB
N_ACC = 10240
ROWS_PT = N_ACC // NS

_MESH = plsc.VectorSubcoreMesh(
    core_axis_name="c", subcore_axis_name="s", num_cores=NC, num_subcores=NS)


@functools.partial(
    pl.kernel,
    out_type=jax.ShapeDtypeStruct((NC, N_ACC), jnp.float32),
    mesh=_MESH,
    scratch_types=[
        pltpu.VMEM((EPB, BATCH), jnp.int32),
        pltpu.VMEM((BATCH,), jnp.float32),
        pltpu.VMEM((ROWS_PT,), jnp.float32),
        pltpu.VMEM_SHARED((N_ACC,), jnp.float32),
    ],
    compiler_params=pltpu.CompilerParams(use_tc_tiling_on_sc=False),
)
def _sc_deg(col_hbm, out_hbm, colb, ones, zb, acc):
    cid = lax.axis_index("c")
    sid = lax.axis_index("s")
    wid = cid * NS + sid
    zero16 = jnp.zeros((16,), jnp.float32)

    def zbody(i, _):
        zb[pl.ds(i * 16, 16)] = zero16
        return 0
    lax.fori_loop(0, ROWS_PT // 16, zbody, 0)
    pltpu.sync_copy(zb, acc.at[pl.ds(sid * ROWS_PT, ROWS_PT)])

    def obody(i, _):
        ones[pl.ds(i * 16, 16)] = zero16 + 1.0
        return 0
    lax.fori_loop(0, BATCH // 16, obody, 0)

    pltpu.sync_copy(col_hbm.at[pl.ds(wid * EPB, EPB)], colb)
    plsc.subcore_barrier()

    def body(j, _):
        pltpu.sync_copy(ones, acc.at[colb.at[j]], add=True)
        return 0
    lax.fori_loop(0, EPB, body, 0)

    plsc.subcore_barrier()
    pltpu.sync_copy(acc.at[pl.ds(sid * ROWS_PT, ROWS_PT)],
                    out_hbm.at[cid, pl.ds(sid * ROWS_PT, ROWS_PT)])


@functools.partial(
    pl.kernel,
    out_type=jax.ShapeDtypeStruct((NC, N_ACC, CP), jnp.float32),
    mesh=_MESH,
    scratch_types=[
        pltpu.VMEM((EPB, BATCH), jnp.int32),
        pltpu.VMEM((EPB, BATCH), jnp.int32),
        pltpu.VMEM((BATCH, CP), jnp.float32),
        pltpu.VMEM((BATCH, CP), jnp.float32),
        pltpu.VMEM((ROWS_PT, CP), jnp.float32),
        pltpu.VMEM_SHARED((N_ACC, CP), jnp.float32),
        pltpu.SemaphoreType.DMA,
        pltpu.SemaphoreType.DMA,
    ],
    compiler_params=pltpu.CompilerParams(use_tc_tiling_on_sc=False),
)
def _sc_hop(z_hbm, row_hbm, col_hbm, out_hbm,
            rowb, colb, gba, gbb, zb, acc, sema, semb):
    cid = lax.axis_index("c")
    sid = lax.axis_index("s")
    wid = cid * NS + sid
    zero16 = jnp.zeros((16,), jnp.float32)

    def zbody(i, _):
        zb[i, pl.ds(0, 16)] = zero16
        zb[i, pl.ds(16, 16)] = zero16
        zb[i, pl.ds(32, 16)] = zero16
        return 0
    lax.fori_loop(0, ROWS_PT, zbody, 0)
    pltpu.sync_copy(zb, acc.at[pl.ds(sid * ROWS_PT, ROWS_PT)])

    pltpu.sync_copy(row_hbm.at[pl.ds(wid * EPB, EPB)], rowb)
    pltpu.sync_copy(col_hbm.at[pl.ds(wid * EPB, EPB)], colb)
    plsc.subcore_barrier()

    ga = pltpu.async_copy(z_hbm.at[rowb.at[0]], gba, sema)

    def body(j2, _):
        ja = 2 * j2
        ga = pltpu.make_async_copy(z_hbm.at[rowb.at[ja]], gba, sema)
        ga.wait()

        @pl.when(ja + 1 < EPB)
        def _():
            pltpu.async_copy(z_hbm.at[rowb.at[ja + 1]], gbb, semb)
        pltpu.sync_copy(gba, acc.at[colb.at[ja]], add=True)

        @pl.when(ja + 1 < EPB)
        def _():
            pltpu.make_async_copy(z_hbm.at[rowb.at[ja + 1]], gbb, semb).wait()

            @pl.when(ja + 2 < EPB)
            def _():
                pltpu.async_copy(z_hbm.at[rowb.at[ja + 2]], gba, sema)
            pltpu.sync_copy(gbb, acc.at[colb.at[ja + 1]], add=True)
        return 0
    lax.fori_loop(0, (EPB + 1) // 2, body, 0)

    plsc.subcore_barrier()
    pltpu.sync_copy(acc.at[pl.ds(sid * ROWS_PT, ROWS_PT)],
                    out_hbm.at[cid, pl.ds(sid * ROWS_PT, ROWS_PT)])


_BLK = 512


def _tc_project(xp, wpt, degp):
    def body(x_ref, w_ref, dg_ref, z_ref):
        deg = dg_ref[0, :] + dg_ref[1, :] + 1.0
        dinv = lax.rsqrt(deg)
        y = jnp.dot(x_ref[...], w_ref[...], preferred_element_type=jnp.float32)
        z_ref[...] = y * dinv[:, None]
    return pl.pallas_call(
        body,
        grid=(N_ACC // _BLK,),
        in_specs=[
            pl.BlockSpec((_BLK, D), lambda i: (i, 0)),
            pl.BlockSpec((D, CP), lambda i: (0, 0)),
            pl.BlockSpec((2, _BLK), lambda i: (0, i)),
        ],
        out_specs=pl.BlockSpec((_BLK, CP), lambda i: (i, 0)),
        out_shape=jax.ShapeDtypeStruct((N_ACC, CP), jnp.float32),
    )(xp, wpt, degp)


def _tc_mid(sa, sb, z0, degp):
    def body(sa_ref, sb_ref, z_ref, dg_ref, o_ref):
        deg = dg_ref[0, :] + dg_ref[1, :] + 1.0
        s = sa_ref[...] + sb_ref[...] + z_ref[...]
        o_ref[...] = s * (1.0 / deg)[:, None]
    return pl.pallas_call(
        body,
        grid=(N_ACC // _BLK,),
        in_specs=[
            pl.BlockSpec((_BLK, CP), lambda i: (i, 0)),
            pl.BlockSpec((_BLK, CP), lambda i: (i, 0)),
            pl.BlockSpec((_BLK, CP), lambda i: (i, 0)),
            pl.BlockSpec((2, _BLK), lambda i: (0, i)),
        ],
        out_specs=pl.BlockSpec((_BLK, CP), lambda i: (i, 0)),
        out_shape=jax.ShapeDtypeStruct((N_ACC, CP), jnp.float32),
    )(sa, sb, z0, degp)


_BLKO = 1024


def _tc_final(sa, sb, z1, degp, bp):
    def body(sa_ref, sb_ref, z_ref, dg_ref, b_ref, o_ref):
        deg = dg_ref[0, :] + dg_ref[1, :] + 1.0
        dinv = lax.rsqrt(deg)
        h = (sa_ref[...] + sb_ref[...] + z_ref[...]) * dinv[:, None] + b_ref[...]
        colmask = lax.broadcasted_iota(jnp.int32, (_BLKO, CP), 1) < C
        hm = jnp.where(colmask, h, -1e30)
        m = jnp.max(hm, axis=1, keepdims=True)
        ex = jnp.where(colmask, jnp.exp(hm - m), 0.0)
        lse = jnp.log(jnp.sum(ex, axis=1, keepdims=True))
        o_ref[...] = hm - m - lse
    return pl.pallas_call(
        body,
        grid=(-(-N // _BLKO),),
        in_specs=[
            pl.BlockSpec((_BLKO, CP), lambda i: (i, 0)),
            pl.BlockSpec((_BLKO, CP), lambda i: (i, 0)),
            pl.BlockSpec((_BLKO, CP), lambda i: (i, 0)),
            pl.BlockSpec((2, _BLKO), lambda i: (0, i)),
            pl.BlockSpec((1, CP), lambda i: (0, 0)),
        ],
        out_specs=pl.BlockSpec((_BLKO, CP), lambda i: (i, 0)),
        out_shape=jax.ShapeDtypeStruct((N, CP), jnp.float32),
    )(sa, sb, z1, degp, bp)


def kernel(x, edge_index, W, b):
    row = edge_index[0]
    col = edge_index[1]
    rowp = jnp.concatenate(
        [row, jnp.zeros((E_PAD - E,), row.dtype)]).reshape(E_PAD // BATCH, BATCH)
    colp = jnp.concatenate(
        [col, jnp.full((E_PAD - E,), N, col.dtype)]).reshape(E_PAD // BATCH, BATCH)
    xp = jnp.zeros((N_ACC, D), x.dtype).at[:N].set(x)
    wpt = jnp.zeros((D, CP), W.dtype).at[:, :C].set(W.T)
    bp = jnp.zeros((1, CP), b.dtype).at[0, :C].set(b)

    degp = _sc_deg(colp)
    z0 = _tc_project(xp, wpt, degp)
    s1 = _sc_hop(z0, rowp, colp)
    z1 = _tc_mid(s1[0], s1[1], z0, degp)
    s2 = _sc_hop(z1, rowp, colp)
    outp = _tc_final(s2[0], s2[1], z1, degp, bp)
    return outp[:, :C]
